# Initial kernel scaffold; baseline (speedup 1.0000x reference)
import jax
import jax.numpy as jnp
from jax.experimental import pallas as pl


def kernel(x, edge_index, batch, W1, b1, W2, b2, W3, b3, Wc, bc):
    raise NotImplementedError("write your pallas kernel here")


# speedup vs baseline: 6.7997x; 6.7997x over previous
import functools

import jax
import jax.numpy as jnp
from jax import lax
from jax.experimental import pallas as pl
from jax.experimental.pallas import tpu as pltpu
from jax.experimental.pallas import tpu_sc as plsc

N_RAW = 10000
N_PAD = 10240
D = 128
G = 64
NC, NS = 2, 16
ROWS_PER_TILE = N_PAD // NS
CHUNK = 128
CHUNKS = 80
E_PAD = NC * NS * CHUNKS * CHUNK
BLK = 1024
GRID = N_PAD // BLK
HIST_W = 128

def _sc_mesh():
    return plsc.VectorSubcoreMesh(core_axis_name="c", subcore_axis_name="s",
                                  num_cores=NC, num_subcores=NS)


def _sc_degree(dst_idx, ones_chunk, zeros_deg):
    @functools.partial(
        pl.kernel,
        out_type=jax.ShapeDtypeStruct((NC, N_PAD, HIST_W), jnp.float32),
        mesh=_sc_mesh(),
        scratch_types=[
            pltpu.VMEM((CHUNKS, CHUNK), jnp.int32),
            pltpu.VMEM((CHUNK, HIST_W), jnp.float32),
            pltpu.VMEM_SHARED((N_PAD, HIST_W), jnp.float32),
        ],
    )
    def k(dst_hbm, ones_hbm, z_hbm, out_hbm, dst_v, ones_v, acc):
        c = lax.axis_index("c")
        s = lax.axis_index("s")
        base = s * ROWS_PER_TILE
        pltpu.sync_copy(z_hbm, acc.at[pl.ds(base, ROWS_PER_TILE)])
        pltpu.sync_copy(dst_hbm.at[c, s], dst_v)
        pltpu.sync_copy(ones_hbm, ones_v)
        plsc.subcore_barrier()

        @pl.loop(0, CHUNKS)
        def _(j):
            pltpu.sync_copy(ones_v, acc.at[dst_v.at[j]], add=True)

        plsc.subcore_barrier()
        pltpu.sync_copy(acc.at[pl.ds(base, ROWS_PER_TILE)],
                        out_hbm.at[c, pl.ds(base, ROWS_PER_TILE)])

    return k(dst_idx, ones_chunk, zeros_deg)


def _sc_edge_scatter(g_rows, src_idx, dst_idx, zeros_rows):
    @functools.partial(
        pl.kernel,
        out_type=jax.ShapeDtypeStruct((NC, N_PAD, D), jnp.float32),
        mesh=_sc_mesh(),
        scratch_types=[
            pltpu.VMEM((CHUNKS, CHUNK), jnp.int32),
            pltpu.VMEM((CHUNKS, CHUNK), jnp.int32),
            pltpu.VMEM((CHUNK, D), jnp.float32),
            pltpu.VMEM_SHARED((N_PAD, D), jnp.float32),
        ],
    )
    def k(g_hbm, src_hbm, dst_hbm, z_hbm, out_hbm, src_v, dst_v, rows_v, acc):
        c = lax.axis_index("c")
        s = lax.axis_index("s")
        base = s * ROWS_PER_TILE
        pltpu.sync_copy(z_hbm, acc.at[pl.ds(base, ROWS_PER_TILE)])
        pltpu.sync_copy(src_hbm.at[c, s], src_v)
        pltpu.sync_copy(dst_hbm.at[c, s], dst_v)
        plsc.subcore_barrier()

        @pl.loop(0, CHUNKS)
        def _(j):
            pltpu.sync_copy(g_hbm.at[src_v.at[j]], rows_v)
            pltpu.sync_copy(rows_v, acc.at[dst_v.at[j]], add=True)

        plsc.subcore_barrier()
        pltpu.sync_copy(acc.at[pl.ds(base, ROWS_PER_TILE)],
                        out_hbm.at[c, pl.ds(base, ROWS_PER_TILE)])

    return k(g_rows, src_idx, dst_idx, zeros_rows)


def _mm(a, w):
    def body(a_ref, w_ref, o_ref):
        o_ref[...] = jnp.dot(a_ref[...], w_ref[...],
                             preferred_element_type=jnp.float32)

    return pl.pallas_call(
        body,
        grid=(GRID,),
        in_specs=[pl.BlockSpec((BLK, D), lambda i: (i, 0)),
                  pl.BlockSpec((D, D), lambda i: (0, 0))],
        out_specs=pl.BlockSpec((BLK, D), lambda i: (i, 0)),
        out_shape=jax.ShapeDtypeStruct((N_PAD, D), jnp.float32),
    )(a, w)


def _prep(h0, h1, z1):
    def body(h0_ref, h1_ref, z_ref, dis_ref, g_ref):
        i = pl.program_id(0)
        deg = h0_ref[:, 0:1] + h1_ref[:, 0:1] + 1.0
        dis = lax.rsqrt(deg)
        row = i * BLK + lax.broadcasted_iota(jnp.int32, (BLK, 1), 0)
        dis = jnp.where(row < N_RAW, dis, 0.0)
        dis_ref[...] = dis
        g_ref[...] = dis * z_ref[...]

    return pl.pallas_call(
        body,
        grid=(GRID,),
        in_specs=[pl.BlockSpec((BLK, HIST_W), lambda i: (i, 0)),
                  pl.BlockSpec((BLK, HIST_W), lambda i: (i, 0)),
                  pl.BlockSpec((BLK, D), lambda i: (i, 0))],
        out_specs=[pl.BlockSpec((BLK, 1), lambda i: (i, 0)),
                   pl.BlockSpec((BLK, D), lambda i: (i, 0))],
        out_shape=[jax.ShapeDtypeStruct((N_PAD, 1), jnp.float32),
                   jax.ShapeDtypeStruct((N_PAD, D), jnp.float32)],
    )(h0, h1, z1)


def _layer(s0, s1, g, dis, b_row, w):
    def body(s0_ref, s1_ref, g_ref, dis_ref, b_ref, w_ref, o_ref):
        dis = dis_ref[...]
        h = (s0_ref[...] + s1_ref[...] + g_ref[...]) * dis + b_ref[...]
        h = jnp.maximum(h, 0.0)
        o_ref[...] = dis * jnp.dot(h, w_ref[...],
                                   preferred_element_type=jnp.float32)

    return pl.pallas_call(
        body,
        grid=(GRID,),
        in_specs=[pl.BlockSpec((BLK, D), lambda i: (i, 0)),
                  pl.BlockSpec((BLK, D), lambda i: (i, 0)),
                  pl.BlockSpec((BLK, D), lambda i: (i, 0)),
                  pl.BlockSpec((BLK, 1), lambda i: (i, 0)),
                  pl.BlockSpec((1, D), lambda i: (0, 0)),
                  pl.BlockSpec((D, D), lambda i: (0, 0))],
        out_specs=pl.BlockSpec((BLK, D), lambda i: (i, 0)),
        out_shape=jax.ShapeDtypeStruct((N_PAD, D), jnp.float32),
    )(s0, s1, g, dis, b_row, w)


def _final(s0, s1, g, dis, b_row, batch_row, wc, bc_row):
    def body(s0_ref, s1_ref, g_ref, dis_ref, b_ref, bt_ref, wc_ref, bc_ref,
             o_ref, accp, accc):
        i = pl.program_id(0)

        @pl.when(i == 0)
        def _():
            accp[...] = jnp.zeros_like(accp)
            accc[...] = jnp.zeros_like(accc)

        dis = dis_ref[...]
        h = (s0_ref[...] + s1_ref[...] + g_ref[...]) * dis + b_ref[...]
        oh = (bt_ref[...] == lax.broadcasted_iota(jnp.int32, (G, BLK), 0)
              ).astype(jnp.float32)
        accp[...] += jnp.dot(oh, h, preferred_element_type=jnp.float32)
        accc[...] += jnp.dot(oh, jnp.ones((BLK, D), jnp.float32),
                             preferred_element_type=jnp.float32)

        @pl.when(i == GRID - 1)
        def _():
            pooled = accp[...] / jnp.maximum(accc[...], 1.0)
            o_ref[...] = jnp.dot(pooled, wc_ref[...],
                                 preferred_element_type=jnp.float32) + bc_ref[...]

    return pl.pallas_call(
        body,
        grid=(GRID,),
        in_specs=[pl.BlockSpec((BLK, D), lambda i: (i, 0)),
                  pl.BlockSpec((BLK, D), lambda i: (i, 0)),
                  pl.BlockSpec((BLK, D), lambda i: (i, 0)),
                  pl.BlockSpec((BLK, 1), lambda i: (i, 0)),
                  pl.BlockSpec((1, D), lambda i: (0, 0)),
                  pl.BlockSpec((1, BLK), lambda i: (0, i)),
                  pl.BlockSpec((D, 2), lambda i: (0, 0)),
                  pl.BlockSpec((1, 2), lambda i: (0, 0))],
        out_specs=pl.BlockSpec((G, 2), lambda i: (0, 0)),
        out_shape=jax.ShapeDtypeStruct((G, 2), jnp.float32),
        scratch_shapes=[pltpu.VMEM((G, D), jnp.float32),
                        pltpu.VMEM((G, D), jnp.float32)],
    )(s0, s1, g, dis, b_row, batch_row, wc, bc_row)


def kernel(x, edge_index, batch, W1, b1, W2, b2, W3, b3, Wc, bc):
    src = edge_index[0].astype(jnp.int32)
    dst = edge_index[1].astype(jnp.int32)
    n_e = src.shape[0]
    pad = jnp.full((E_PAD - n_e,), N_RAW, jnp.int32)
    src_t = jnp.concatenate([src, pad]).reshape(NC, NS, CHUNKS, CHUNK)
    dst_t = jnp.concatenate([dst, pad]).reshape(NC, NS, CHUNKS, CHUNK)

    x_p = jnp.pad(x.astype(jnp.float32), ((0, N_PAD - N_RAW), (0, 0)))
    batch_row = jnp.concatenate(
        [batch.astype(jnp.int32), jnp.full((N_PAD - N_RAW,), G, jnp.int32)]
    ).reshape(1, N_PAD)

    zeros_rows = jnp.zeros((ROWS_PER_TILE, D), jnp.float32)
    zeros_deg = jnp.zeros((ROWS_PER_TILE, HIST_W), jnp.float32)
    ones_chunk = jnp.ones((CHUNK, HIST_W), jnp.float32)

    hist = _sc_degree(dst_t, ones_chunk, zeros_deg)
    z1 = _mm(x_p, W1)
    dis, g = _prep(hist[0], hist[1], z1)

    for w_next, b_this in ((W2, b1), (W3, b2)):
        s = _sc_edge_scatter(g, src_t, dst_t, zeros_rows)
        g = _layer(s[0], s[1], g, dis, b_this.reshape(1, D), w_next)

    s = _sc_edge_scatter(g, src_t, dst_t, zeros_rows)
    return _final(s[0], s[1], g, dis, b3.reshape(1, D), batch_row,
                  Wc, bc.reshape(1, 2))


# speedup vs baseline: 7.6047x; 1.1184x over previous
import functools

import jax
import jax.numpy as jnp
from jax import lax
from jax.experimental import pallas as pl
from jax.experimental.pallas import tpu as pltpu
from jax.experimental.pallas import tpu_sc as plsc

N = 10000
N_ACC = 10112
N_GT = 10112
D = 128
G = 64
NC, NS = 2, 16
RPT = N_ACC // NS
CHUNK = 64
CHUNKS = 160
PHASES = 4
CPP = CHUNKS // PHASES
E_PAD = NC * NS * CHUNKS * CHUNK
NBUF = 4
BLK = 1000
GRID = N // BLK
HIST_W = 128


def _sc_mesh():
    return plsc.VectorSubcoreMesh(core_axis_name="c", subcore_axis_name="s",
                                  num_cores=NC, num_subcores=NS)


def _sc_degree(dst_idx, ones_chunk, zeros_slab):
    @functools.partial(
        pl.kernel,
        out_type=jax.ShapeDtypeStruct((NC, N_ACC, HIST_W), jnp.float32),
        mesh=_sc_mesh(),
        scratch_types=[
            pltpu.VMEM((CHUNKS, CHUNK), jnp.int32),
            pltpu.VMEM((CHUNK, HIST_W), jnp.float32),
            pltpu.VMEM_SHARED((N_ACC, HIST_W), jnp.float32),
        ],
    )
    def k(dst_hbm, ones_hbm, z_hbm, out_hbm, dst_v, ones_v, acc):
        c = lax.axis_index("c")
        s = lax.axis_index("s")
        base = s * RPT
        pltpu.sync_copy(z_hbm, acc.at[pl.ds(base, RPT)])
        pltpu.sync_copy(dst_hbm.at[c, s], dst_v)
        pltpu.sync_copy(ones_hbm, ones_v)
        plsc.subcore_barrier()

        @pl.loop(0, CHUNKS)
        def _(j):
            pltpu.sync_copy(ones_v, acc.at[dst_v.at[j]], add=True)

        plsc.subcore_barrier()
        pltpu.sync_copy(acc.at[pl.ds(base, RPT)],
                        out_hbm.at[c, pl.ds(base, RPT)])

    return k(dst_idx, ones_chunk, zeros_slab)


def _sc_edge_scatter(g_rows, src_idx, dst_idx, zeros_slab):
    @functools.partial(
        pl.kernel,
        out_type=jax.ShapeDtypeStruct((NC, N_ACC, D), jnp.float32),
        mesh=_sc_mesh(),
        scratch_types=[
            pltpu.VMEM((CPP, CHUNK), jnp.int32),
            pltpu.VMEM((CPP, CHUNK), jnp.int32),
            pltpu.VMEM((NBUF, CHUNK, D), jnp.float32),
            pltpu.VMEM_SHARED((N_ACC, D), jnp.float32),
            pltpu.SemaphoreType.DMA((NBUF,)),
        ],
    )
    def k(g_hbm, src_hbm, dst_hbm, z_hbm, out_hbm, src_v, dst_v, rows_all,
          acc, semg):
        rows = [rows_all.at[b] for b in range(NBUF)]
        c = lax.axis_index("c")
        s = lax.axis_index("s")
        base = s * RPT
        pltpu.sync_copy(z_hbm, acc.at[pl.ds(base, RPT)])
        plsc.subcore_barrier()

        def gather(j, b):
            pltpu.async_copy(g_hbm.at[src_v.at[j]], rows[b], semg.at[b])

        def gather_wait(j, b):
            pltpu.make_async_copy(g_hbm.at[src_v.at[j]], rows[b],
                                  semg.at[b]).wait()

        for p in range(PHASES):
            pltpu.sync_copy(src_hbm.at[c, s, pl.ds(p * CPP, CPP)], src_v)
            pltpu.sync_copy(dst_hbm.at[c, s, pl.ds(p * CPP, CPP)], dst_v)

            for b in range(NBUF):
                gather(b, b)

            @pl.loop(0, CPP, step=NBUF)
            def _(jj):
                for b in range(NBUF):
                    gather_wait(jj + b, b)
                    pltpu.sync_copy(rows[b], acc.at[dst_v.at[jj + b]],
                                    add=True)

                    @pl.when(jj + NBUF < CPP)
                    def _(b=b):
                        gather(jj + NBUF + b, b)

        plsc.subcore_barrier()
        pltpu.sync_copy(acc.at[pl.ds(base, RPT)],
                        out_hbm.at[c, pl.ds(base, RPT)])

    return k(g_rows, src_idx, dst_idx, zeros_slab)


def _mm(a, w):
    def body(a_ref, w_ref, o_ref):
        o_ref[...] = jnp.dot(a_ref[...], w_ref[...],
                             preferred_element_type=jnp.float32)

    return pl.pallas_call(
        body,
        grid=(GRID,),
        in_specs=[pl.BlockSpec((BLK, D), lambda i: (i, 0)),
                  pl.BlockSpec((D, D), lambda i: (0, 0))],
        out_specs=pl.BlockSpec((BLK, D), lambda i: (i, 0)),
        out_shape=jax.ShapeDtypeStruct((N, D), jnp.float32),
    )(a, w)


def _prep(h0, h1, z1):
    def body(h0_ref, h1_ref, z_ref, dis_ref, g_ref):
        deg = h0_ref[:, 0:1] + h1_ref[:, 0:1] + 1.0
        dis = lax.rsqrt(deg)
        dis_ref[...] = dis
        g_ref[...] = dis * z_ref[...]

    return pl.pallas_call(
        body,
        grid=(GRID,),
        in_specs=[pl.BlockSpec((BLK, HIST_W), lambda i: (i, 0)),
                  pl.BlockSpec((BLK, HIST_W), lambda i: (i, 0)),
                  pl.BlockSpec((BLK, D), lambda i: (i, 0))],
        out_specs=[pl.BlockSpec((BLK, 1), lambda i: (i, 0)),
                   pl.BlockSpec((BLK, D), lambda i: (i, 0))],
        out_shape=[jax.ShapeDtypeStruct((N, 1), jnp.float32),
                   jax.ShapeDtypeStruct((N, D), jnp.float32)],
    )(h0, h1, z1)


def _layer(s0, s1, g, dis, b_row, w):
    def body(s0_ref, s1_ref, g_ref, dis_ref, b_ref, w_ref, o_ref):
        dis = dis_ref[...]
        h = (s0_ref[...] + s1_ref[...] + g_ref[...]) * dis + b_ref[...]
        h = jnp.maximum(h, 0.0)
        o_ref[...] = dis * jnp.dot(h, w_ref[...],
                                   preferred_element_type=jnp.float32)

    return pl.pallas_call(
        body,
        grid=(GRID,),
        in_specs=[pl.BlockSpec((BLK, D), lambda i: (i, 0)),
                  pl.BlockSpec((BLK, D), lambda i: (i, 0)),
                  pl.BlockSpec((BLK, D), lambda i: (i, 0)),
                  pl.BlockSpec((BLK, 1), lambda i: (i, 0)),
                  pl.BlockSpec((1, D), lambda i: (0, 0)),
                  pl.BlockSpec((D, D), lambda i: (0, 0))],
        out_specs=pl.BlockSpec((BLK, D), lambda i: (i, 0)),
        out_shape=jax.ShapeDtypeStruct((N, D), jnp.float32),
    )(s0, s1, g, dis, b_row, w)


def _final(s0, s1, g, dis, b_row, batch_row, wc, bc_row):
    def body(s0_ref, s1_ref, g_ref, dis_ref, b_ref, bt_ref, wc_ref, bc_ref,
             o_ref, accp, accc):
        i = pl.program_id(0)

        @pl.when(i == 0)
        def _():
            accp[...] = jnp.zeros_like(accp)
            accc[...] = jnp.zeros_like(accc)

        dis = dis_ref[...]
        h = (s0_ref[...] + s1_ref[...] + g_ref[...]) * dis + b_ref[...]
        oh = (bt_ref[0] == lax.broadcasted_iota(jnp.int32, (G, BLK), 0)
              ).astype(jnp.float32)
        accp[...] += jnp.dot(oh, h, preferred_element_type=jnp.float32)
        accc[...] += jnp.dot(oh, jnp.ones((BLK, D), jnp.float32),
                             preferred_element_type=jnp.float32)

        @pl.when(i == GRID - 1)
        def _():
            pooled = accp[...] / jnp.maximum(accc[...], 1.0)
            o_ref[...] = jnp.dot(pooled, wc_ref[...],
                                 preferred_element_type=jnp.float32) + bc_ref[...]

    return pl.pallas_call(
        body,
        grid=(GRID,),
        in_specs=[pl.BlockSpec((BLK, D), lambda i: (i, 0)),
                  pl.BlockSpec((BLK, D), lambda i: (i, 0)),
                  pl.BlockSpec((BLK, D), lambda i: (i, 0)),
                  pl.BlockSpec((BLK, 1), lambda i: (i, 0)),
                  pl.BlockSpec((1, D), lambda i: (0, 0)),
                  pl.BlockSpec((1, 1, BLK), lambda i: (i, 0, 0)),
                  pl.BlockSpec((D, 2), lambda i: (0, 0)),
                  pl.BlockSpec((1, 2), lambda i: (0, 0))],
        out_specs=pl.BlockSpec((G, 2), lambda i: (0, 0)),
        out_shape=jax.ShapeDtypeStruct((G, 2), jnp.float32),
        scratch_shapes=[pltpu.VMEM((G, D), jnp.float32),
                        pltpu.VMEM((G, D), jnp.float32)],
    )(s0, s1, g, dis, b_row, batch_row, wc, bc_row)


def kernel(x, edge_index, batch, W1, b1, W2, b2, W3, b3, Wc, bc):
    src = edge_index[0].astype(jnp.int32)
    dst = edge_index[1].astype(jnp.int32)
    n_e = src.shape[0]
    pad = jnp.full((E_PAD - n_e,), N, jnp.int32)
    src_t = jnp.concatenate([src, pad]).reshape(NC, NS, CHUNKS, CHUNK)
    dst_t = jnp.concatenate([dst, pad]).reshape(NC, NS, CHUNKS, CHUNK)

    batch_row = batch.astype(jnp.int32).reshape(GRID, 1, BLK)

    zeros_slab = jnp.zeros((RPT, D), jnp.float32)
    ones_chunk = jnp.ones((CHUNK, HIST_W), jnp.float32)

    hist = _sc_degree(dst_t, ones_chunk, zeros_slab)
    z1 = _mm(x.astype(jnp.float32), W1)
    dis, g = _prep(hist[0, :N], hist[1, :N], z1)

    for w_next, b_this in ((W2, b1), (W3, b2)):
        g_t = jnp.pad(g, ((0, N_GT - N), (0, 0)))
        s = _sc_edge_scatter(g_t, src_t, dst_t, zeros_slab)
        g = _layer(s[0, :N], s[1, :N], g, dis, b_this.reshape(1, D), w_next)

    g_t = jnp.pad(g, ((0, N_GT - N), (0, 0)))
    s = _sc_edge_scatter(g_t, src_t, dst_t, zeros_slab)
    return _final(s[0, :N], s[1, :N], g, dis, b3.reshape(1, D), batch_row,
                  Wc, bc.reshape(1, 2))


# speedup vs baseline: 8.2208x; 1.0810x over previous
import functools

import jax
import jax.numpy as jnp
from jax import lax
from jax.experimental import pallas as pl
from jax.experimental.pallas import tpu as pltpu
from jax.experimental.pallas import tpu_sc as plsc

N = 10000
N_ACC = 10112
N_GT = 10112
D = 128
G = 64
NC, NS = 2, 16
RPT = N_ACC // NS
CHUNK = 64
CHUNKS = 160
PHASES = 4
CPP = CHUNKS // PHASES
E_PAD = NC * NS * CHUNKS * CHUNK
NBUF = 4
BLK = 1000
GRID = N // BLK
HIST_W = 128


def _sc_mesh():
    return plsc.VectorSubcoreMesh(core_axis_name="c", subcore_axis_name="s",
                                  num_cores=NC, num_subcores=NS)


def _sc_degree(dst_idx, ones_chunk, zeros_slab):
    @functools.partial(
        pl.kernel,
        out_type=jax.ShapeDtypeStruct((NC, N_ACC, HIST_W), jnp.float32),
        mesh=_sc_mesh(),
        scratch_types=[
            pltpu.VMEM((CHUNKS, CHUNK), jnp.int32),
            pltpu.VMEM((CHUNK, HIST_W), jnp.float32),
            pltpu.VMEM_SHARED((N_ACC, HIST_W), jnp.float32),
        ],
    )
    def k(dst_hbm, ones_hbm, z_hbm, out_hbm, dst_v, ones_v, acc):
        c = lax.axis_index("c")
        s = lax.axis_index("s")
        base = s * RPT
        pltpu.sync_copy(z_hbm, acc.at[pl.ds(base, RPT)])
        pltpu.sync_copy(dst_hbm.at[c, s], dst_v)
        pltpu.sync_copy(ones_hbm, ones_v)
        plsc.subcore_barrier()

        @pl.loop(0, CHUNKS)
        def _(j):
            pltpu.sync_copy(ones_v, acc.at[dst_v.at[j]], add=True)

        plsc.subcore_barrier()
        pltpu.sync_copy(acc.at[pl.ds(base, RPT)],
                        out_hbm.at[c, pl.ds(base, RPT)])

    return k(dst_idx, ones_chunk, zeros_slab)


def _sc_edge_scatter(g_rows, src_idx, dst_idx, zeros_slab):
    @functools.partial(
        pl.kernel,
        out_type=jax.ShapeDtypeStruct((NC, N_ACC, D), jnp.float32),
        mesh=_sc_mesh(),
        scratch_types=[
            pltpu.VMEM((CPP, CHUNK), jnp.int32),
            pltpu.VMEM((CPP, CHUNK), jnp.int32),
            pltpu.VMEM((NBUF, CHUNK, D), jnp.float32),
            pltpu.VMEM_SHARED((N_ACC, D), jnp.float32),
            pltpu.SemaphoreType.DMA((NBUF,)),
        ],
    )
    def k(g_hbm, src_hbm, dst_hbm, z_hbm, out_hbm, src_v, dst_v, rows_all,
          acc, semg):
        rows = [rows_all.at[b] for b in range(NBUF)]
        c = lax.axis_index("c")
        s = lax.axis_index("s")
        base = s * RPT
        pltpu.sync_copy(z_hbm, acc.at[pl.ds(base, RPT)])
        plsc.subcore_barrier()

        def gather(j, b):
            pltpu.async_copy(g_hbm.at[src_v.at[j]], rows[b], semg.at[b])

        def gather_wait(j, b):
            pltpu.make_async_copy(g_hbm.at[src_v.at[j]], rows[b],
                                  semg.at[b]).wait()

        for p in range(PHASES):
            pltpu.sync_copy(src_hbm.at[c, s, pl.ds(p * CPP, CPP)], src_v)
            pltpu.sync_copy(dst_hbm.at[c, s, pl.ds(p * CPP, CPP)], dst_v)

            for b in range(NBUF):
                gather(b, b)

            @pl.loop(0, CPP, step=NBUF)
            def _(jj):
                for b in range(NBUF):
                    gather_wait(jj + b, b)
                    pltpu.sync_copy(rows[b], acc.at[dst_v.at[jj + b]],
                                    add=True)

                    @pl.when(jj + NBUF < CPP)
                    def _(b=b):
                        gather(jj + NBUF + b, b)

        plsc.subcore_barrier()
        pltpu.sync_copy(acc.at[pl.ds(base, RPT)],
                        out_hbm.at[c, pl.ds(base, RPT)])

    return k(g_rows, src_idx, dst_idx, zeros_slab)


def _mm(a, w):
    def body(a_ref, w_ref, o_ref):
        o_ref[...] = jnp.dot(a_ref[...], w_ref[...],
                             preferred_element_type=jnp.float32)

    return pl.pallas_call(
        body,
        grid=(GRID,),
        in_specs=[pl.BlockSpec((BLK, D), lambda i: (i, 0)),
                  pl.BlockSpec((D, D), lambda i: (0, 0))],
        out_specs=pl.BlockSpec((BLK, D), lambda i: (i, 0)),
        out_shape=jax.ShapeDtypeStruct((N, D), jnp.float32),
    )(a, w)


def _prep(h0, h1, z1):
    def body(h0_ref, h1_ref, z_ref, dis_ref, g_ref):
        deg = h0_ref[:, 0:1] + h1_ref[:, 0:1] + 1.0
        dis = lax.rsqrt(deg)
        dis_ref[...] = dis
        g_ref[...] = dis * z_ref[...]

    return pl.pallas_call(
        body,
        grid=(GRID,),
        in_specs=[pl.BlockSpec((BLK, HIST_W), lambda i: (i, 0)),
                  pl.BlockSpec((BLK, HIST_W), lambda i: (i, 0)),
                  pl.BlockSpec((BLK, D), lambda i: (i, 0))],
        out_specs=[pl.BlockSpec((BLK, 1), lambda i: (i, 0)),
                   pl.BlockSpec((BLK, D), lambda i: (i, 0))],
        out_shape=[jax.ShapeDtypeStruct((N, 1), jnp.float32),
                   jax.ShapeDtypeStruct((N, D), jnp.float32)],
    )(h0, h1, z1)


def _layer(s0, s1, g, dis, b_row, w):
    def body(s0_ref, s1_ref, g_ref, dis_ref, b_ref, w_ref, o_ref):
        dis = dis_ref[...]
        h = (s0_ref[...] + s1_ref[...] + g_ref[...]) * dis + b_ref[...]
        h = jnp.maximum(h, 0.0)
        o_ref[...] = dis * jnp.dot(h, w_ref[...],
                                   preferred_element_type=jnp.float32)

    return pl.pallas_call(
        body,
        grid=(GRID,),
        in_specs=[pl.BlockSpec((BLK, D), lambda i: (i, 0)),
                  pl.BlockSpec((BLK, D), lambda i: (i, 0)),
                  pl.BlockSpec((BLK, D), lambda i: (i, 0)),
                  pl.BlockSpec((BLK, 1), lambda i: (i, 0)),
                  pl.BlockSpec((1, D), lambda i: (0, 0)),
                  pl.BlockSpec((D, D), lambda i: (0, 0))],
        out_specs=pl.BlockSpec((BLK, D), lambda i: (i, 0)),
        out_shape=jax.ShapeDtypeStruct((N, D), jnp.float32),
    )(s0, s1, g, dis, b_row, w)


def _final(s0, s1, g, dis, b_row, batch_row, wc, bc_row):
    def body(s0_ref, s1_ref, g_ref, dis_ref, b_ref, bt_ref, wc_ref, bc_ref,
             o_ref, accp, accc):
        i = pl.program_id(0)

        @pl.when(i == 0)
        def _():
            accp[...] = jnp.zeros_like(accp)
            accc[...] = jnp.zeros_like(accc)

        dis = dis_ref[...]
        h = (s0_ref[...] + s1_ref[...] + g_ref[...]) * dis + b_ref[...]
        oh = (bt_ref[0] == lax.broadcasted_iota(jnp.int32, (G, BLK), 0)
              ).astype(jnp.float32)
        accp[...] += jnp.dot(oh, h, preferred_element_type=jnp.float32)
        accc[...] += jnp.dot(oh, jnp.ones((BLK, D), jnp.float32),
                             preferred_element_type=jnp.float32)

        @pl.when(i == GRID - 1)
        def _():
            pooled = accp[...] / jnp.maximum(accc[...], 1.0)
            o_ref[...] = jnp.dot(pooled, wc_ref[...],
                                 preferred_element_type=jnp.float32) + bc_ref[...]

    return pl.pallas_call(
        body,
        grid=(GRID,),
        in_specs=[pl.BlockSpec((BLK, D), lambda i: (i, 0)),
                  pl.BlockSpec((BLK, D), lambda i: (i, 0)),
                  pl.BlockSpec((BLK, D), lambda i: (i, 0)),
                  pl.BlockSpec((BLK, 1), lambda i: (i, 0)),
                  pl.BlockSpec((1, D), lambda i: (0, 0)),
                  pl.BlockSpec((1, 1, BLK), lambda i: (i, 0, 0)),
                  pl.BlockSpec((D, 2), lambda i: (0, 0)),
                  pl.BlockSpec((1, 2), lambda i: (0, 0))],
        out_specs=pl.BlockSpec((G, 2), lambda i: (0, 0)),
        out_shape=jax.ShapeDtypeStruct((G, 2), jnp.float32),
        scratch_shapes=[pltpu.VMEM((G, D), jnp.float32),
                        pltpu.VMEM((G, D), jnp.float32)],
    )(s0, s1, g, dis, b_row, batch_row, wc, bc_row)


def kernel(x, edge_index, batch, W1, b1, W2, b2, W3, b3, Wc, bc):
    src = edge_index[0].astype(jnp.int32)
    dst = edge_index[1].astype(jnp.int32)
    n_e = src.shape[0]
    pad = jnp.full((E_PAD - n_e,), N, jnp.int32)
    src_t = jnp.concatenate([src, pad]).reshape(NC, NS, CHUNKS, CHUNK)
    dst_t = jnp.concatenate([dst, pad]).reshape(NC, NS, CHUNKS, CHUNK)
    src_t = src_t + jnp.arange(NC, dtype=jnp.int32).reshape(NC, 1, 1, 1) * N_GT

    batch_row = batch.astype(jnp.int32).reshape(GRID, 1, BLK)

    zeros_slab = jnp.zeros((RPT, D), jnp.float32)
    ones_chunk = jnp.ones((CHUNK, HIST_W), jnp.float32)

    hist = _sc_degree(dst_t, ones_chunk, zeros_slab)
    z1 = _mm(x.astype(jnp.float32), W1)
    dis, g = _prep(hist[0, :N], hist[1, :N], z1)

    for w_next, b_this in ((W2, b1), (W3, b2)):
        g_t = jnp.pad(g, ((0, N_GT - N), (0, 0)))
        s = _sc_edge_scatter(jnp.concatenate([g_t, g_t]), src_t, dst_t,
                             zeros_slab)
        g = _layer(s[0, :N], s[1, :N], g, dis, b_this.reshape(1, D), w_next)

    g_t = jnp.pad(g, ((0, N_GT - N), (0, 0)))
    s = _sc_edge_scatter(jnp.concatenate([g_t, g_t]), src_t, dst_t, zeros_slab)
    return _final(s[0, :N], s[1, :N], g, dis, b3.reshape(1, D), batch_row,
                  Wc, bc.reshape(1, 2))


# speedup vs baseline: 22.4828x; 2.7349x over previous
import functools

import jax
import jax.numpy as jnp
from jax import lax
from jax.experimental import pallas as pl
from jax.experimental.pallas import tpu as pltpu
from jax.experimental.pallas import tpu_sc as plsc

N = 10000
N_ACC = 10624
N_GT = 10624
D = 128
G = 64
NC, NS = 2, 16
RPT = N_ACC // NS
CHUNK = 64
CHUNKS = 160
PHASES = 4
CPP = CHUNKS // PHASES
E_PAD = NC * NS * CHUNKS * CHUNK
NBUF = 4
BLK = 1000
GRID = N // BLK
HIST_W = 128


def _sc_mesh():
    return plsc.VectorSubcoreMesh(core_axis_name="c", subcore_axis_name="s",
                                  num_cores=NC, num_subcores=NS)


def _sc_degree(dst_idx, ones_chunk, zeros_slab):
    @functools.partial(
        pl.kernel,
        out_type=jax.ShapeDtypeStruct((NC, N_ACC, HIST_W), jnp.float32),
        mesh=_sc_mesh(),
        scratch_types=[
            pltpu.VMEM((CHUNKS, CHUNK), jnp.int32),
            pltpu.VMEM((CHUNK, HIST_W), jnp.float32),
            pltpu.VMEM_SHARED((N_ACC, HIST_W), jnp.float32),
        ],
    )
    def k(dst_hbm, ones_hbm, z_hbm, out_hbm, dst_v, ones_v, acc):
        c = lax.axis_index("c")
        s = lax.axis_index("s")
        base = s * RPT
        pltpu.sync_copy(z_hbm, acc.at[pl.ds(base, RPT)])
        pltpu.sync_copy(dst_hbm.at[c, s], dst_v)
        pltpu.sync_copy(ones_hbm, ones_v)
        plsc.subcore_barrier()

        @pl.loop(0, CHUNKS)
        def _(j):
            pltpu.sync_copy(ones_v, acc.at[dst_v.at[j]], add=True)

        plsc.subcore_barrier()
        pltpu.sync_copy(acc.at[pl.ds(base, RPT)],
                        out_hbm.at[c, pl.ds(base, RPT)])

    return k(dst_idx, ones_chunk, zeros_slab)


def _sc_edge_scatter(g_rows, src_idx, dst_idx, zeros_slab):
    @functools.partial(
        pl.kernel,
        out_type=jax.ShapeDtypeStruct((NC, N_ACC, D), jnp.float32),
        mesh=_sc_mesh(),
        scratch_types=[
            pltpu.VMEM((CPP, CHUNK), jnp.int32),
            pltpu.VMEM((CPP, CHUNK), jnp.int32),
            pltpu.VMEM((NBUF, CHUNK, D), jnp.float32),
            pltpu.VMEM_SHARED((N_ACC, D), jnp.float32),
            pltpu.SemaphoreType.DMA((NBUF,)),
        ],
    )
    def k(g_hbm, src_hbm, dst_hbm, z_hbm, out_hbm, src_v, dst_v, rows_all,
          acc, semg):
        rows = [rows_all.at[b] for b in range(NBUF)]
        c = lax.axis_index("c")
        s = lax.axis_index("s")
        base = s * RPT
        pltpu.sync_copy(z_hbm, acc.at[pl.ds(base, RPT)])
        plsc.subcore_barrier()

        def gather(j, b):
            pltpu.async_copy(g_hbm.at[src_v.at[j]], rows[b], semg.at[b])

        def gather_wait(j, b):
            pltpu.make_async_copy(g_hbm.at[src_v.at[j]], rows[b],
                                  semg.at[b]).wait()

        for p in range(PHASES):
            pltpu.sync_copy(src_hbm.at[c, s, pl.ds(p * CPP, CPP)], src_v)
            pltpu.sync_copy(dst_hbm.at[c, s, pl.ds(p * CPP, CPP)], dst_v)

            for b in range(NBUF):
                gather(b, b)

            @pl.loop(0, CPP, step=NBUF)
            def _(jj):
                for b in range(NBUF):
                    gather_wait(jj + b, b)
                    pltpu.sync_copy(rows[b], acc.at[dst_v.at[jj + b]],
                                    add=True)

                    @pl.when(jj + NBUF < CPP)
                    def _(b=b):
                        gather(jj + NBUF + b, b)

        plsc.subcore_barrier()
        pltpu.sync_copy(acc.at[pl.ds(base, RPT)],
                        out_hbm.at[c, pl.ds(base, RPT)])

    return k(g_rows, src_idx, dst_idx, zeros_slab)


def _mm(a, w):
    def body(a_ref, w_ref, o_ref):
        o_ref[...] = jnp.dot(a_ref[...], w_ref[...],
                             preferred_element_type=jnp.float32)

    return pl.pallas_call(
        body,
        grid=(GRID,),
        in_specs=[pl.BlockSpec((BLK, D), lambda i: (i, 0)),
                  pl.BlockSpec((D, D), lambda i: (0, 0))],
        out_specs=pl.BlockSpec((BLK, D), lambda i: (i, 0)),
        out_shape=jax.ShapeDtypeStruct((N, D), jnp.float32),
    )(a, w)


def _prep(h0, h1, z1):
    def body(h0_ref, h1_ref, z_ref, dis_ref, g_ref):
        deg = h0_ref[:, 0:1] + h1_ref[:, 0:1] + 1.0
        dis = lax.rsqrt(deg)
        dis_ref[...] = dis
        g_ref[...] = dis * z_ref[...]

    return pl.pallas_call(
        body,
        grid=(GRID,),
        in_specs=[pl.BlockSpec((BLK, HIST_W), lambda i: (i, 0)),
                  pl.BlockSpec((BLK, HIST_W), lambda i: (i, 0)),
                  pl.BlockSpec((BLK, D), lambda i: (i, 0))],
        out_specs=[pl.BlockSpec((BLK, 1), lambda i: (i, 0)),
                   pl.BlockSpec((BLK, D), lambda i: (i, 0))],
        out_shape=[jax.ShapeDtypeStruct((N, 1), jnp.float32),
                   jax.ShapeDtypeStruct((N, D), jnp.float32)],
    )(h0, h1, z1)


def _layer(s0, s1, g, dis, b_row, w):
    def body(s0_ref, s1_ref, g_ref, dis_ref, b_ref, w_ref, o_ref):
        dis = dis_ref[...]
        h = (s0_ref[...] + s1_ref[...] + g_ref[...]) * dis + b_ref[...]
        h = jnp.maximum(h, 0.0)
        o_ref[...] = dis * jnp.dot(h, w_ref[...],
                                   preferred_element_type=jnp.float32)

    return pl.pallas_call(
        body,
        grid=(GRID,),
        in_specs=[pl.BlockSpec((BLK, D), lambda i: (i, 0)),
                  pl.BlockSpec((BLK, D), lambda i: (i, 0)),
                  pl.BlockSpec((BLK, D), lambda i: (i, 0)),
                  pl.BlockSpec((BLK, 1), lambda i: (i, 0)),
                  pl.BlockSpec((1, D), lambda i: (0, 0)),
                  pl.BlockSpec((D, D), lambda i: (0, 0))],
        out_specs=pl.BlockSpec((BLK, D), lambda i: (i, 0)),
        out_shape=jax.ShapeDtypeStruct((N, D), jnp.float32),
    )(s0, s1, g, dis, b_row, w)


def _final(s0, s1, g, dis, b_row, batch_row, wc, bc_row):
    def body(s0_ref, s1_ref, g_ref, dis_ref, b_ref, bt_ref, wc_ref, bc_ref,
             o_ref, accp, accc):
        i = pl.program_id(0)

        @pl.when(i == 0)
        def _():
            accp[...] = jnp.zeros_like(accp)
            accc[...] = jnp.zeros_like(accc)

        dis = dis_ref[...]
        h = (s0_ref[...] + s1_ref[...] + g_ref[...]) * dis + b_ref[...]
        oh = (bt_ref[0] == lax.broadcasted_iota(jnp.int32, (G, BLK), 0)
              ).astype(jnp.float32)
        accp[...] += jnp.dot(oh, h, preferred_element_type=jnp.float32)
        accc[...] += jnp.dot(oh, jnp.ones((BLK, D), jnp.float32),
                             preferred_element_type=jnp.float32)

        @pl.when(i == GRID - 1)
        def _():
            pooled = accp[...] / jnp.maximum(accc[...], 1.0)
            o_ref[...] = jnp.dot(pooled, wc_ref[...],
                                 preferred_element_type=jnp.float32) + bc_ref[...]

    return pl.pallas_call(
        body,
        grid=(GRID,),
        in_specs=[pl.BlockSpec((BLK, D), lambda i: (i, 0)),
                  pl.BlockSpec((BLK, D), lambda i: (i, 0)),
                  pl.BlockSpec((BLK, D), lambda i: (i, 0)),
                  pl.BlockSpec((BLK, 1), lambda i: (i, 0)),
                  pl.BlockSpec((1, D), lambda i: (0, 0)),
                  pl.BlockSpec((1, 1, BLK), lambda i: (i, 0, 0)),
                  pl.BlockSpec((D, 2), lambda i: (0, 0)),
                  pl.BlockSpec((1, 2), lambda i: (0, 0))],
        out_specs=pl.BlockSpec((G, 2), lambda i: (0, 0)),
        out_shape=jax.ShapeDtypeStruct((G, 2), jnp.float32),
        scratch_shapes=[pltpu.VMEM((G, D), jnp.float32),
                        pltpu.VMEM((G, D), jnp.float32)],
    )(s0, s1, g, dis, b_row, batch_row, wc, bc_row)


def kernel(x, edge_index, batch, W1, b1, W2, b2, W3, b3, Wc, bc):
    src = edge_index[0].astype(jnp.int32)
    dst = edge_index[1].astype(jnp.int32)
    n_e = src.shape[0]
    pad = N + (jnp.arange(E_PAD - n_e, dtype=jnp.int32) % (N_ACC - N))
    src_t = jnp.concatenate([src, pad]).reshape(NC, NS, CHUNKS, CHUNK)
    dst_t = jnp.concatenate([dst, pad]).reshape(NC, NS, CHUNKS, CHUNK)
    src_t = src_t + jnp.arange(NC, dtype=jnp.int32).reshape(NC, 1, 1, 1) * N_GT

    batch_row = batch.astype(jnp.int32).reshape(GRID, 1, BLK)

    zeros_slab = jnp.zeros((RPT, D), jnp.float32)
    ones_chunk = jnp.ones((CHUNK, HIST_W), jnp.float32)

    hist = _sc_degree(dst_t, ones_chunk, zeros_slab)
    z1 = _mm(x.astype(jnp.float32), W1)
    dis, g = _prep(hist[0, :N], hist[1, :N], z1)

    for w_next, b_this in ((W2, b1), (W3, b2)):
        g_t = jnp.pad(g, ((0, N_GT - N), (0, 0)))
        s = _sc_edge_scatter(jnp.concatenate([g_t, g_t]), src_t, dst_t,
                             zeros_slab)
        g = _layer(s[0, :N], s[1, :N], g, dis, b_this.reshape(1, D), w_next)

    g_t = jnp.pad(g, ((0, N_GT - N), (0, 0)))
    s = _sc_edge_scatter(jnp.concatenate([g_t, g_t]), src_t, dst_t, zeros_slab)
    return _final(s[0, :N], s[1, :N], g, dis, b3.reshape(1, D), batch_row,
                  Wc, bc.reshape(1, 2))


# speedup vs baseline: 23.4872x; 1.0447x over previous
import functools

import jax
import jax.numpy as jnp
from jax import lax
from jax.experimental import pallas as pl
from jax.experimental.pallas import tpu as pltpu
from jax.experimental.pallas import tpu_sc as plsc

N = 10000
N_ACC = 10624
N_GT = 10624
D = 128
G = 64
NC, NS = 2, 16
RPT = N_ACC // NS
CHUNK = 64
CHUNKS = 160
PHASES = 4
CPP = CHUNKS // PHASES
E_PAD = NC * NS * CHUNKS * CHUNK
NBUF = 4
BLK = 1000
GRID = N // BLK
HIST_W = 128


def _sc_mesh():
    return plsc.VectorSubcoreMesh(core_axis_name="c", subcore_axis_name="s",
                                  num_cores=NC, num_subcores=NS)


def _sc_degree(dst_idx, ones_chunk, zeros_slab):
    @functools.partial(
        pl.kernel,
        out_type=jax.ShapeDtypeStruct((NC, N_ACC, HIST_W), jnp.float32),
        mesh=_sc_mesh(),
        scratch_types=[
            pltpu.VMEM((CHUNKS, CHUNK), jnp.int32),
            pltpu.VMEM((CHUNK, HIST_W), jnp.float32),
            pltpu.VMEM_SHARED((N_ACC, HIST_W), jnp.float32),
        ],
    )
    def k(dst_hbm, ones_hbm, z_hbm, out_hbm, dst_v, ones_v, acc):
        c = lax.axis_index("c")
        s = lax.axis_index("s")
        base = s * RPT
        pltpu.sync_copy(z_hbm, acc.at[pl.ds(base, RPT)])
        pltpu.sync_copy(dst_hbm.at[c, s], dst_v)
        pltpu.sync_copy(ones_hbm, ones_v)
        plsc.subcore_barrier()

        @pl.loop(0, CHUNKS)
        def _(j):
            pltpu.sync_copy(ones_v, acc.at[dst_v.at[j]], add=True)

        plsc.subcore_barrier()
        pltpu.sync_copy(acc.at[pl.ds(base, RPT)],
                        out_hbm.at[c, pl.ds(base, RPT)])

    return k(dst_idx, ones_chunk, zeros_slab)


def _sc_edge_scatter(g_rows, src_idx, dst_idx, zeros_slab):
    @functools.partial(
        pl.kernel,
        out_type=jax.ShapeDtypeStruct((NC, N_ACC, D), jnp.float32),
        mesh=_sc_mesh(),
        scratch_types=[
            pltpu.VMEM((CPP, CHUNK), jnp.int32),
            pltpu.VMEM((CPP, CHUNK), jnp.int32),
            pltpu.VMEM((NBUF, CHUNK, D), jnp.float32),
            pltpu.VMEM_SHARED((N_ACC, D), jnp.float32),
            pltpu.SemaphoreType.DMA((NBUF,)),
        ],
    )
    def k(g_hbm, src_hbm, dst_hbm, z_hbm, out_hbm, src_v, dst_v, rows_all,
          acc, semg):
        rows = [rows_all.at[b] for b in range(NBUF)]
        c = lax.axis_index("c")
        s = lax.axis_index("s")
        base = s * RPT
        pltpu.sync_copy(z_hbm, acc.at[pl.ds(base, RPT)])
        plsc.subcore_barrier()

        def gather(j, b):
            pltpu.async_copy(g_hbm.at[src_v.at[j]], rows[b], semg.at[b])

        def gather_wait(j, b):
            pltpu.make_async_copy(g_hbm.at[src_v.at[j]], rows[b],
                                  semg.at[b]).wait()

        for p in range(PHASES):
            pltpu.sync_copy(src_hbm.at[c, s, pl.ds(p * CPP, CPP)], src_v)
            pltpu.sync_copy(dst_hbm.at[c, s, pl.ds(p * CPP, CPP)], dst_v)

            for b in range(NBUF):
                gather(b, b)

            @pl.loop(0, CPP, step=NBUF)
            def _(jj):
                for b in range(NBUF):
                    gather_wait(jj + b, b)
                    pltpu.sync_copy(rows[b], acc.at[dst_v.at[jj + b]],
                                    add=True)

                    @pl.when(jj + NBUF < CPP)
                    def _(b=b):
                        gather(jj + NBUF + b, b)

        plsc.subcore_barrier()
        pltpu.sync_copy(acc.at[pl.ds(base, RPT)],
                        out_hbm.at[c, pl.ds(base, RPT)])

    return k(g_rows, src_idx, dst_idx, zeros_slab)


def _mm(a, w):
    def body(a_ref, w_ref, o_ref):
        o_ref[...] = jnp.dot(a_ref[...], w_ref[...],
                             preferred_element_type=jnp.float32)

    return pl.pallas_call(
        body,
        grid=(GRID,),
        in_specs=[pl.BlockSpec((BLK, D), lambda i: (i, 0)),
                  pl.BlockSpec((D, D), lambda i: (0, 0))],
        out_specs=pl.BlockSpec((BLK, D), lambda i: (i, 0)),
        out_shape=jax.ShapeDtypeStruct((N, D), jnp.float32),
    )(a, w)


def _prep(h0, h1, z1):
    def body(h0_ref, h1_ref, z_ref, dis_ref, g_ref):
        deg = h0_ref[:, 0:1] + h1_ref[:, 0:1] + 1.0
        dis = lax.rsqrt(deg)
        dis_ref[...] = dis
        g_ref[...] = dis * z_ref[...]

    return pl.pallas_call(
        body,
        grid=(GRID,),
        in_specs=[pl.BlockSpec((BLK, HIST_W), lambda i: (i, 0)),
                  pl.BlockSpec((BLK, HIST_W), lambda i: (i, 0)),
                  pl.BlockSpec((BLK, D), lambda i: (i, 0))],
        out_specs=[pl.BlockSpec((BLK, 1), lambda i: (i, 0)),
                   pl.BlockSpec((BLK, D), lambda i: (i, 0))],
        out_shape=[jax.ShapeDtypeStruct((N, 1), jnp.float32),
                   jax.ShapeDtypeStruct((N, D), jnp.float32)],
    )(h0, h1, z1)


def _layer(s0, s1, g, dis, b_row, w):
    def body(s0_ref, s1_ref, g_ref, dis_ref, b_ref, w_ref, o_ref):
        dis = dis_ref[...]
        h = (s0_ref[...] + s1_ref[...] + g_ref[...]) * dis + b_ref[...]
        h = jnp.maximum(h, 0.0)
        o_ref[...] = dis * jnp.dot(h, w_ref[...],
                                   preferred_element_type=jnp.float32)

    return pl.pallas_call(
        body,
        grid=(GRID,),
        in_specs=[pl.BlockSpec((BLK, D), lambda i: (i, 0)),
                  pl.BlockSpec((BLK, D), lambda i: (i, 0)),
                  pl.BlockSpec((BLK, D), lambda i: (i, 0)),
                  pl.BlockSpec((BLK, 1), lambda i: (i, 0)),
                  pl.BlockSpec((1, D), lambda i: (0, 0)),
                  pl.BlockSpec((D, D), lambda i: (0, 0))],
        out_specs=pl.BlockSpec((BLK, D), lambda i: (i, 0)),
        out_shape=jax.ShapeDtypeStruct((N, D), jnp.float32),
    )(s0, s1, g, dis, b_row, w)


def _final(s0, s1, g, dis, b_row, batch_row, wc, bc_row):
    def body(s0_ref, s1_ref, g_ref, dis_ref, b_ref, bt_ref, wc_ref, bc_ref,
             o_ref, accp, accc):
        i = pl.program_id(0)

        @pl.when(i == 0)
        def _():
            accp[...] = jnp.zeros_like(accp)
            accc[...] = jnp.zeros_like(accc)

        dis = dis_ref[...]
        h = (s0_ref[...] + s1_ref[...] + g_ref[...]) * dis + b_ref[...]
        oh = (bt_ref[0] == lax.broadcasted_iota(jnp.int32, (G, BLK), 0)
              ).astype(jnp.float32)
        accp[...] += jnp.dot(oh, h, preferred_element_type=jnp.float32)
        accc[...] += jnp.dot(oh, jnp.ones((BLK, D), jnp.float32),
                             preferred_element_type=jnp.float32)

        @pl.when(i == GRID - 1)
        def _():
            pooled = accp[...] / jnp.maximum(accc[...], 1.0)
            o_ref[...] = jnp.dot(pooled, wc_ref[...],
                                 preferred_element_type=jnp.float32) + bc_ref[...]

    return pl.pallas_call(
        body,
        grid=(GRID,),
        in_specs=[pl.BlockSpec((BLK, D), lambda i: (i, 0)),
                  pl.BlockSpec((BLK, D), lambda i: (i, 0)),
                  pl.BlockSpec((BLK, D), lambda i: (i, 0)),
                  pl.BlockSpec((BLK, 1), lambda i: (i, 0)),
                  pl.BlockSpec((1, D), lambda i: (0, 0)),
                  pl.BlockSpec((1, 1, BLK), lambda i: (i, 0, 0)),
                  pl.BlockSpec((D, 2), lambda i: (0, 0)),
                  pl.BlockSpec((1, 2), lambda i: (0, 0))],
        out_specs=pl.BlockSpec((G, 2), lambda i: (0, 0)),
        out_shape=jax.ShapeDtypeStruct((G, 2), jnp.float32),
        scratch_shapes=[pltpu.VMEM((G, D), jnp.float32),
                        pltpu.VMEM((G, D), jnp.float32)],
    )(s0, s1, g, dis, b_row, batch_row, wc, bc_row)


def kernel(x, edge_index, batch, W1, b1, W2, b2, W3, b3, Wc, bc):
    src = edge_index[0].astype(jnp.int32)
    dst = edge_index[1].astype(jnp.int32)
    n_e = src.shape[0]
    pad = N + (jnp.arange(E_PAD - n_e, dtype=jnp.int32) % (N_ACC - N))
    src_t = jnp.concatenate([src, pad]).reshape(NC, NS, CHUNKS, CHUNK)
    dst_t = jnp.concatenate([dst, pad]).reshape(NC, NS, CHUNKS, CHUNK)

    batch_row = batch.astype(jnp.int32).reshape(GRID, 1, BLK)

    zeros_slab = jnp.zeros((RPT, D), jnp.float32)
    ones_chunk = jnp.ones((CHUNK, HIST_W), jnp.float32)

    hist = _sc_degree(dst_t, ones_chunk, zeros_slab)
    z1 = _mm(x.astype(jnp.float32), W1)
    dis, g = _prep(hist[0, :N], hist[1, :N], z1)

    for w_next, b_this in ((W2, b1), (W3, b2)):
        g_t = jnp.pad(g, ((0, N_GT - N), (0, 0)))
        s = _sc_edge_scatter(g_t, src_t, dst_t, zeros_slab)
        g = _layer(s[0, :N], s[1, :N], g, dis, b_this.reshape(1, D), w_next)

    g_t = jnp.pad(g, ((0, N_GT - N), (0, 0)))
    s = _sc_edge_scatter(g_t, src_t, dst_t, zeros_slab)
    return _final(s[0, :N], s[1, :N], g, dis, b3.reshape(1, D), batch_row,
                  Wc, bc.reshape(1, 2))


# speedup vs baseline: 24.5278x; 1.0443x over previous
import functools

import jax
import jax.numpy as jnp
from jax import lax
from jax.experimental import pallas as pl
from jax.experimental.pallas import tpu as pltpu
from jax.experimental.pallas import tpu_sc as plsc

N = 10000
N_ACC = 10624
N_GT = 10624
D = 128
G = 64
NC, NS = 2, 16
RPT = N_ACC // NS
CHUNK = 64
CHUNKS = 160
PHASES = 4
CPP = CHUNKS // PHASES
E_PAD = NC * NS * CHUNKS * CHUNK
NBUF = 4
BLK = 1000
GRID = N // BLK
HIST_W = 128


def _sc_mesh():
    return plsc.VectorSubcoreMesh(core_axis_name="c", subcore_axis_name="s",
                                  num_cores=NC, num_subcores=NS)


def _sc_degree(dst_idx, ones_chunk, zeros_slab):
    @functools.partial(
        pl.kernel,
        out_type=jax.ShapeDtypeStruct((NC, N_ACC, HIST_W), jnp.float32),
        mesh=_sc_mesh(),
        scratch_types=[
            pltpu.VMEM((CHUNKS, CHUNK), jnp.int32),
            pltpu.VMEM((CHUNK, HIST_W), jnp.float32),
            pltpu.VMEM_SHARED((N_ACC, HIST_W), jnp.float32),
        ],
    )
    def k(dst_hbm, ones_hbm, z_hbm, out_hbm, dst_v, ones_v, acc):
        c = lax.axis_index("c")
        s = lax.axis_index("s")
        base = s * RPT
        pltpu.sync_copy(z_hbm, acc.at[pl.ds(base, RPT)])
        pltpu.sync_copy(dst_hbm.at[c, s], dst_v)
        pltpu.sync_copy(ones_hbm, ones_v)
        plsc.subcore_barrier()

        @pl.loop(0, CHUNKS)
        def _(j):
            pltpu.sync_copy(ones_v, acc.at[dst_v.at[j]], add=True)

        plsc.subcore_barrier()
        pltpu.sync_copy(acc.at[pl.ds(base, RPT)],
                        out_hbm.at[c, pl.ds(base, RPT)])

    return k(dst_idx, ones_chunk, zeros_slab)


def _sc_edge_scatter(g_rows, src_idx, dst_idx, zeros_slab):
    @functools.partial(
        pl.kernel,
        out_type=jax.ShapeDtypeStruct((NC, N_ACC, D), jnp.float32),
        mesh=_sc_mesh(),
        scratch_types=[
            pltpu.VMEM((CPP, CHUNK), jnp.int32),
            pltpu.VMEM((CPP, CHUNK), jnp.int32),
            pltpu.VMEM((NBUF, CHUNK, D), jnp.float32),
            pltpu.VMEM_SHARED((N_ACC, D), jnp.float32),
            pltpu.SemaphoreType.DMA((NBUF,)),
        ],
    )
    def k(g_hbm, src_hbm, dst_hbm, z_hbm, out_hbm, src_v, dst_v, rows_all,
          acc, semg):
        rows = [rows_all.at[b] for b in range(NBUF)]
        c = lax.axis_index("c")
        s = lax.axis_index("s")
        base = s * RPT
        pltpu.sync_copy(z_hbm, acc.at[pl.ds(base, RPT)])
        plsc.subcore_barrier()

        def gather(j, b):
            pltpu.async_copy(g_hbm.at[src_v.at[j]], rows[b], semg.at[b])

        def gather_wait(j, b):
            pltpu.make_async_copy(g_hbm.at[src_v.at[j]], rows[b],
                                  semg.at[b]).wait()

        for p in range(PHASES):
            pltpu.sync_copy(src_hbm.at[c, s, pl.ds(p * CPP, CPP)], src_v)
            pltpu.sync_copy(dst_hbm.at[c, s, pl.ds(p * CPP, CPP)], dst_v)

            for b in range(NBUF):
                gather(b, b)

            @pl.loop(0, CPP, step=NBUF)
            def _(jj):
                for b in range(NBUF):
                    gather_wait(jj + b, b)
                    pltpu.sync_copy(rows[b], acc.at[dst_v.at[jj + b]],
                                    add=True)

                    @pl.when(jj + NBUF < CPP)
                    def _(b=b):
                        gather(jj + NBUF + b, b)

        plsc.subcore_barrier()
        pltpu.sync_copy(acc.at[pl.ds(base, RPT)],
                        out_hbm.at[c, pl.ds(base, RPT)])

    return k(g_rows, src_idx, dst_idx, zeros_slab)


def _mm(a, w):
    def body(a_ref, w_ref, o_ref):
        o_ref[...] = jnp.dot(a_ref[...], w_ref[...],
                             preferred_element_type=jnp.float32)

    return pl.pallas_call(
        body,
        grid=(GRID,),
        in_specs=[pl.BlockSpec((BLK, D), lambda i: (i, 0)),
                  pl.BlockSpec((D, D), lambda i: (0, 0))],
        out_specs=pl.BlockSpec((BLK, D), lambda i: (i, 0)),
        out_shape=jax.ShapeDtypeStruct((N, D), jnp.float32),
    )(a, w)


def _prep(h0, h1, z1):
    def body(h0_ref, h1_ref, z_ref, dis_ref, g_ref):
        deg = h0_ref[:, 0:1] + h1_ref[:, 0:1] + 1.0
        dis = lax.rsqrt(deg)
        dis_ref[...] = dis
        g_ref[...] = dis * z_ref[...]

    return pl.pallas_call(
        body,
        grid=(GRID,),
        in_specs=[pl.BlockSpec((BLK, HIST_W), lambda i: (i, 0)),
                  pl.BlockSpec((BLK, HIST_W), lambda i: (i, 0)),
                  pl.BlockSpec((BLK, D), lambda i: (i, 0))],
        out_specs=[pl.BlockSpec((BLK, 1), lambda i: (i, 0)),
                   pl.BlockSpec((BLK, D), lambda i: (i, 0))],
        out_shape=[jax.ShapeDtypeStruct((N, 1), jnp.float32),
                   jax.ShapeDtypeStruct((N, D), jnp.float32)],
    )(h0, h1, z1)


def _layer(s0, s1, g, dis, b_row, w):
    def body(s0_ref, s1_ref, g_ref, dis_ref, b_ref, w_ref, o_ref):
        dis = dis_ref[...]
        h = (s0_ref[0] + s1_ref[0] + g_ref[...]) * dis + b_ref[...]
        h = jnp.maximum(h, 0.0)
        o_ref[...] = dis * jnp.dot(h, w_ref[...],
                                   preferred_element_type=jnp.float32)

    return pl.pallas_call(
        body,
        grid=(GRID,),
        in_specs=[pl.BlockSpec((1, BLK, D), lambda i: (0, i, 0)),
                  pl.BlockSpec((1, BLK, D), lambda i: (1, i, 0)),
                  pl.BlockSpec((BLK, D), lambda i: (i, 0)),
                  pl.BlockSpec((BLK, 1), lambda i: (i, 0)),
                  pl.BlockSpec((1, D), lambda i: (0, 0)),
                  pl.BlockSpec((D, D), lambda i: (0, 0))],
        out_specs=pl.BlockSpec((BLK, D), lambda i: (i, 0)),
        out_shape=jax.ShapeDtypeStruct((N, D), jnp.float32),
    )(s0, s1, g, dis, b_row, w)


def _final(s0, s1, g, dis, b_row, batch_row, wc, bc_row):
    def body(s0_ref, s1_ref, g_ref, dis_ref, b_ref, bt_ref, wc_ref, bc_ref,
             o_ref, accp, accc):
        i = pl.program_id(0)

        @pl.when(i == 0)
        def _():
            accp[...] = jnp.zeros_like(accp)
            accc[...] = jnp.zeros_like(accc)

        dis = dis_ref[...]
        h = (s0_ref[0] + s1_ref[0] + g_ref[...]) * dis + b_ref[...]
        oh = (bt_ref[0] == lax.broadcasted_iota(jnp.int32, (G, BLK), 0)
              ).astype(jnp.float32)
        accp[...] += jnp.dot(oh, h, preferred_element_type=jnp.float32)
        accc[...] += jnp.dot(oh, jnp.ones((BLK, D), jnp.float32),
                             preferred_element_type=jnp.float32)

        @pl.when(i == GRID - 1)
        def _():
            pooled = accp[...] / jnp.maximum(accc[...], 1.0)
            o_ref[...] = jnp.dot(pooled, wc_ref[...],
                                 preferred_element_type=jnp.float32) + bc_ref[...]

    return pl.pallas_call(
        body,
        grid=(GRID,),
        in_specs=[pl.BlockSpec((1, BLK, D), lambda i: (0, i, 0)),
                  pl.BlockSpec((1, BLK, D), lambda i: (1, i, 0)),
                  pl.BlockSpec((BLK, D), lambda i: (i, 0)),
                  pl.BlockSpec((BLK, 1), lambda i: (i, 0)),
                  pl.BlockSpec((1, D), lambda i: (0, 0)),
                  pl.BlockSpec((1, 1, BLK), lambda i: (i, 0, 0)),
                  pl.BlockSpec((D, 2), lambda i: (0, 0)),
                  pl.BlockSpec((1, 2), lambda i: (0, 0))],
        out_specs=pl.BlockSpec((G, 2), lambda i: (0, 0)),
        out_shape=jax.ShapeDtypeStruct((G, 2), jnp.float32),
        scratch_shapes=[pltpu.VMEM((G, D), jnp.float32),
                        pltpu.VMEM((G, D), jnp.float32)],
    )(s0, s1, g, dis, b_row, batch_row, wc, bc_row)


def kernel(x, edge_index, batch, W1, b1, W2, b2, W3, b3, Wc, bc):
    src = edge_index[0].astype(jnp.int32)
    dst = edge_index[1].astype(jnp.int32)
    n_e = src.shape[0]
    pad = N + (jnp.arange(E_PAD - n_e, dtype=jnp.int32) % (N_ACC - N))
    src_t = jnp.concatenate([src, pad]).reshape(NC, NS, CHUNKS, CHUNK)
    dst_t = jnp.concatenate([dst, pad]).reshape(NC, NS, CHUNKS, CHUNK)

    batch_row = batch.astype(jnp.int32).reshape(GRID, 1, BLK)

    zeros_slab = jnp.zeros((RPT, D), jnp.float32)
    ones_chunk = jnp.ones((CHUNK, HIST_W), jnp.float32)

    hist = _sc_degree(dst_t, ones_chunk, zeros_slab)
    z1 = _mm(x.astype(jnp.float32), W1)
    dis, g = _prep(hist[0, :N], hist[1, :N], z1)

    for w_next, b_this in ((W2, b1), (W3, b2)):
        g_t = jnp.pad(g, ((0, N_GT - N), (0, 0)))
        s = _sc_edge_scatter(g_t, src_t, dst_t, zeros_slab)
        g = _layer(s, s, g, dis, b_this.reshape(1, D), w_next)

    g_t = jnp.pad(g, ((0, N_GT - N), (0, 0)))
    s = _sc_edge_scatter(g_t, src_t, dst_t, zeros_slab)
    return _final(s, s, g, dis, b3.reshape(1, D), batch_row,
                  Wc, bc.reshape(1, 2))
